# Initial kernel scaffold; baseline (speedup 1.0000x reference)
import jax
import jax.numpy as jnp
from jax.experimental import pallas as pl


def kernel(pred_logits, pred_boxes, priors, target_sizes):
    raise NotImplementedError("write your pallas kernel here")


# speedup vs baseline: 4.8261x; 4.8261x over previous
import jax
import jax.numpy as jnp
from jax.experimental import pallas as pl
from jax.experimental.pallas import tpu as pltpu

_VAR0 = 0.1
_VAR1 = 0.2
_SCORE_TH = 0.05
_NMS_TH = 0.45
_DETS = 100
_PRE = 1000
_PAD = 1024


def _score_body(logits_ref, bbox_ref, priors_ref, masked_ref, boxes_ref):
    logits = logits_ref[0]
    bbox = bbox_ref[0]
    priors = priors_ref[...]
    pcx = priors[:, 0:1]
    pcy = priors[:, 1:2]
    pw = priors[:, 2:3]
    ph = priors[:, 3:4]
    cx = pcx + bbox[:, 0:1] * _VAR0 * pw
    cy = pcy + bbox[:, 1:2] * _VAR0 * ph
    w = pw * jnp.exp(bbox[:, 2:3] * _VAR1)
    h = ph * jnp.exp(bbox[:, 3:4] * _VAR1)
    x1 = cx - w / 2.0
    y1 = cy - h / 2.0
    x2 = cx + w / 2.0
    y2 = cy + h / 2.0
    boxes_ref[0, :, :] = jnp.concatenate([x1, y1, x2, y2], axis=1)
    bw = x2 - x1
    bh = y2 - y1
    m = jnp.max(logits, axis=1, keepdims=True)
    e = jnp.exp(logits - m)
    s = jnp.sum(e, axis=1, keepdims=True)
    prob = e / s
    scores = prob[:, 1:]
    valid = (scores > _SCORE_TH) & (bw >= 0.01) & (bh >= 0.01)
    masked_ref[0, :, :] = jnp.where(valid, scores, -1.0)


def _nms_body(bc_ref, br_ref, vc_ref, vr_ref, keep_ref, iou_ref):
    bc = bc_ref[0]
    br = br_ref[0]
    vc = vc_ref[0]
    vr = vr_ref[0]
    x1c = bc[:, 0:1]
    y1c = bc[:, 1:2]
    x2c = bc[:, 2:3]
    y2c = bc[:, 3:4]
    x1r = br[0:1, :]
    y1r = br[1:2, :]
    x2r = br[2:3, :]
    y2r = br[3:4, :]
    area_c = jnp.clip(x2c - x1c, 0.0) * jnp.clip(y2c - y1c, 0.0)
    area_r = jnp.clip(x2r - x1r, 0.0) * jnp.clip(y2r - y1r, 0.0)
    iw = jnp.clip(jnp.minimum(x2c, x2r) - jnp.maximum(x1c, x1r), 0.0)
    ih = jnp.clip(jnp.minimum(y2c, y2r) - jnp.maximum(y1c, y1r), 0.0)
    inter = iw * ih
    union = area_c + area_r - inter
    iou_ref[...] = (inter / jnp.maximum(union, 1e-9)) * (vc * vr)

    col_ids = jax.lax.broadcasted_iota(jnp.int32, (1, _PAD), 1)

    def body(i, keep):
        row = iou_ref[pl.ds(i, 1), :]
        keep_i = jnp.sum(jnp.where(col_ids == i, keep, 0.0))
        sup = (row > _NMS_TH) & (col_ids > i) & (keep_i > 0.0)
        return jnp.where(sup, 0.0, keep)

    keep = jax.lax.fori_loop(0, _PRE, body, vr)
    keep_ref[0, :, :] = keep


def kernel(pred_logits, pred_boxes, priors, target_sizes):
    B, P, C = pred_logits.shape
    masked, boxes = pl.pallas_call(
        _score_body,
        grid=(B,),
        in_specs=[
            pl.BlockSpec((1, P, C), lambda b: (b, 0, 0)),
            pl.BlockSpec((1, P, 4), lambda b: (b, 0, 0)),
            pl.BlockSpec((P, 4), lambda b: (0, 0)),
        ],
        out_specs=[
            pl.BlockSpec((1, P, C - 1), lambda b: (b, 0, 0)),
            pl.BlockSpec((1, P, 4), lambda b: (b, 0, 0)),
        ],
        out_shape=[
            jax.ShapeDtypeStruct((B, P, C - 1), jnp.float32),
            jax.ShapeDtypeStruct((B, P, 4), jnp.float32),
        ],
    )(pred_logits, pred_boxes, priors)

    flat = masked.reshape(B, P * (C - 1))
    vals, idx = jax.lax.top_k(flat, _PRE)
    pidx = idx // (C - 1)
    clabels = idx % (C - 1) + 1
    cboxes = jnp.take_along_axis(boxes, pidx[..., None], axis=1)
    cvalid = vals > 0.0
    off = clabels.astype(jnp.float32) * (
        jnp.max(cboxes, axis=(1, 2), keepdims=True)[..., 0] + 1.0
    )
    nboxes = cboxes + off[..., None]

    pad = _PAD - _PRE
    nb = jnp.pad(nboxes, ((0, 0), (0, pad), (0, 0)))
    vf = jnp.pad(cvalid.astype(jnp.float32), ((0, 0), (0, pad)))
    nbt = jnp.transpose(nb, (0, 2, 1))
    vcol = vf[..., None]
    vrow = vf[:, None, :]

    keep = pl.pallas_call(
        _nms_body,
        grid=(B,),
        in_specs=[
            pl.BlockSpec((1, _PAD, 4), lambda b: (b, 0, 0)),
            pl.BlockSpec((1, 4, _PAD), lambda b: (b, 0, 0)),
            pl.BlockSpec((1, _PAD, 1), lambda b: (b, 0, 0)),
            pl.BlockSpec((1, 1, _PAD), lambda b: (b, 0, 0)),
        ],
        out_specs=pl.BlockSpec((1, 1, _PAD), lambda b: (b, 0, 0)),
        out_shape=jax.ShapeDtypeStruct((B, 1, _PAD), jnp.float32),
        scratch_shapes=[pltpu.VMEM((_PAD, _PAD), jnp.float32)],
    )(nb, nbt, vcol, vrow)

    kept = keep[:, 0, :_PRE] > 0.0
    final_masked = jnp.where(kept & cvalid, vals, -1.0)
    fvals, fidx = jax.lax.top_k(final_masked, _DETS)
    ok = fvals > 0.0
    fb = jnp.where(
        ok[..., None], jnp.take_along_axis(cboxes, fidx[..., None], axis=1), 0.0
    )
    fs = jnp.where(ok, fvals, 0.0)
    fl = jnp.where(ok, jnp.take_along_axis(clabels, fidx, axis=1), 0)
    ts = target_sizes[:, ::-1].astype(jnp.float32)
    fb = fb * jnp.concatenate([ts, ts], axis=1)[:, None, :]
    return fb, fs, fl
